# baseline (speedup 1.0000x reference)
import jax
import jax.numpy as jnp
from jax import lax
from jax.experimental import pallas as pl
from jax.experimental.pallas import tpu as pltpu
from jax.experimental.pallas import tpu_sc as plsc

N = 10000
IN_DIM = 128
H_DIM = 128
Z_DIM = 64
HW = Z_DIM // 2

NP = 10112
E = 320000
EA = E + N
CH = 128
NSUB = 16
CPT = 164
EP = NSUB * CPT * CH
RPT = NP // 16
NPH = 1280
RPTH = NPH // 16
NHP = 10016
RHPT = NHP // 16

_BLK = 1024


def _gelu(x):
    return 0.5 * x * (1.0 + jax.lax.erf(x * 0.7071067811865476))


def _tc1_body(x_ref, w1_ref, b1_ref, w2_ref, b2_ref, gw_ref, as_ref, ad_ref,
              h_ref, s_ref, d_ref):
    z = _gelu(jnp.dot(x_ref[...], w1_ref[...],
                      preferred_element_type=jnp.float32) + b1_ref[...])
    z = _gelu(jnp.dot(z, w2_ref[...],
                      preferred_element_type=jnp.float32) + b2_ref[...])
    h = jnp.dot(z, gw_ref[...], preferred_element_type=jnp.float32)
    h_ref[...] = h
    s_ref[...] = jnp.sum(h * as_ref[...], axis=1)
    d_ref[...] = jnp.sum(h * ad_ref[...], axis=1)


def _tc1(X, W1, b1, W2, b2, gW, a_s, a_d):
    grid = ((N + _BLK - 1) // _BLK,)
    return pl.pallas_call(
        _tc1_body,
        grid=grid,
        in_specs=[
            pl.BlockSpec((_BLK, IN_DIM), lambda i: (i, 0)),
            pl.BlockSpec((IN_DIM, H_DIM), lambda i: (0, 0)),
            pl.BlockSpec((H_DIM,), lambda i: (0,)),
            pl.BlockSpec((H_DIM, H_DIM), lambda i: (0, 0)),
            pl.BlockSpec((H_DIM,), lambda i: (0,)),
            pl.BlockSpec((H_DIM, Z_DIM), lambda i: (0, 0)),
            pl.BlockSpec((1, Z_DIM), lambda i: (0, 0)),
            pl.BlockSpec((1, Z_DIM), lambda i: (0, 0)),
        ],
        out_specs=[
            pl.BlockSpec((_BLK, Z_DIM), lambda i: (i, 0)),
            pl.BlockSpec((_BLK,), lambda i: (i,)),
            pl.BlockSpec((_BLK,), lambda i: (i,)),
        ],
        out_shape=[
            jax.ShapeDtypeStruct((N, Z_DIM), jnp.float32),
            jax.ShapeDtypeStruct((N,), jnp.float32),
            jax.ShapeDtypeStruct((N,), jnp.float32),
        ],
    )(X, W1, b1, W2, b2, gW, a_s.reshape(1, Z_DIM), a_d.reshape(1, Z_DIM))


def _sc_gat_body(pk_h, asrc_h, adst_h, shift_h, hp_h,
                 out_h, den_h,
                 pk_v, srcr, dstr, dst2r, asrc_v, adst_v, shift_v,
                 hbrows, frows, erows,
                 out_sp, den_sp, hp_sp, g_sem, s_sem):
    cid = lax.axis_index("c")
    sid = lax.axis_index("s")
    rowbase = sid * RPT

    pltpu.sync_copy(asrc_h, asrc_v)
    pltpu.sync_copy(adst_h, adst_v)
    pltpu.sync_copy(shift_h, shift_v)
    pltpu.sync_copy(pk_h.at[sid], pk_v)
    pltpu.sync_copy(hp_h.at[cid].at[pl.ds(sid * RHPT, RHPT)],
                    hp_sp.at[pl.ds(sid * RHPT, RHPT)])

    def zrow(r, carry):
        for b in range(2):
            for g in range(HW // 16):
                frows[b, r, pl.ds(g * 16, 16)] = jnp.zeros((16,), jnp.float32)
            erows[b, r, :] = jnp.zeros((16,), jnp.float32)
        return carry
    lax.fori_loop(0, CH, zrow, 0)

    def zcopy(k, carry):
        pltpu.sync_copy(frows.at[0].at[pl.ds(0, 128)],
                        out_sp.at[pl.ds(rowbase + k * 128, 128)])
        return carry
    lax.fori_loop(0, 4, zcopy, 0)
    pltpu.sync_copy(frows.at[0].at[pl.ds(0, 120)],
                    out_sp.at[pl.ds(rowbase + 512, 120)])
    rowbase_h = sid * RPTH
    pltpu.sync_copy(erows.at[0].at[pl.ds(0, RPTH)],
                    den_sp.at[pl.ds(rowbase_h, RPTH)])

    plsc.subcore_barrier()

    zero16 = jnp.zeros((16,), jnp.int32)
    lane = lax.iota(jnp.int32, 16)

    def decode(ck, b):
        def d_body(j, c2):
            sl = pl.ds(j * 16, 16)
            p = pk_v[ck, sl]
            d16 = lax.shift_right_logical(p, 14)
            srcr[b, sl] = p & 16383
            dstr[b, sl] = d16
            dst2r[b, sl] = lax.shift_right_logical(d16, 3)
            return c2
        lax.fori_loop(0, CH // 16, d_body, 0)

    def e_compute(b, er_ref):
        def e_body(j, c2):
            s16 = srcr[b, pl.ds(j * 16, 16)]
            d16 = dstr[b, pl.ds(j * 16, 16)]
            a = plsc.load_gather(asrc_v, [s16]) + plsc.load_gather(adst_v, [d16])
            a = jnp.maximum(a, 0.2 * a)
            e16 = jnp.exp(a - plsc.load_gather(shift_v, [d16]))
            p2 = (d16 & 7) * 2
            zf = jnp.zeros((16,), jnp.float32)
            row = j * 16 + lane
            for q in range(2, 16, 2):
                plsc.store_scatter(er_ref, [row, (p2 + q) & 14], zf)
            plsc.store_scatter(er_ref, [row, p2], e16)
            return c2
        lax.fori_loop(0, CH // 16, e_body, 0)

    def mult(b, er_ref):
        def m_body(i, c2):
            for k in range(2):
                r = 2 * i + k
                er = plsc.load_gather(
                    er_ref, [jnp.full((16,), r, jnp.int32), zero16])
                ea, eb = plsc.unpack(hbrows[b, r, :],
                                     format=plsc.PackFormat.INTERLEAVED)
                frows[b, r, pl.ds(0, 16)] = ea * er
                frows[b, r, pl.ds(16, 16)] = eb * er
            return c2
        lax.fori_loop(0, CH // 2, m_body, 0)

    def issue_gather(b):
        pltpu.async_copy(hp_sp.at[srcr.at[b]], hbrows.at[b], g_sem)

    def wait_gather(b):
        pltpu.make_async_copy(hp_sp.at[srcr.at[b]], hbrows.at[b],
                              g_sem).wait()

    def issue_scatter(b):
        pltpu.async_copy(frows.at[b], out_sp.at[dstr.at[b]], s_sem, add=True)
        pltpu.async_copy(erows.at[b], den_sp.at[dst2r.at[b]], s_sem,
                         add=True)

    def wait_scatter(b):
        pltpu.make_async_copy(frows.at[b], out_sp.at[dstr.at[b]],
                              s_sem).wait()
        pltpu.make_async_copy(erows.at[b], den_sp.at[dst2r.at[b]],
                              s_sem).wait()

    decode(0, 0)
    issue_gather(0)

    def chunk_body(ck, carry):
        b = ck & 1

        @pl.when(ck > 0)
        def _():
            wait_scatter(1 - b)

        @pl.when(ck + 1 < CPT)
        def _():
            decode(ck + 1, 1 - b)
            issue_gather(1 - b)
        e_compute(b, erows.at[b])
        wait_gather(b)
        mult(b, erows.at[b])
        issue_scatter(b)
        return carry
    lax.fori_loop(0, CPT, chunk_body, 0)
    wait_scatter(1)

    plsc.subcore_barrier()
    pltpu.sync_copy(out_sp.at[pl.ds(rowbase, RPT)],
                    out_h.at[cid].at[pl.ds(rowbase, RPT)])
    pltpu.sync_copy(den_sp.at[pl.ds(rowbase_h, RPTH)],
                    den_h.at[cid].at[pl.ds(rowbase_h, RPTH)])


def _sc_gat(pk3d, asrc, adst, shift, hp2):
    f = pl.kernel(
        _sc_gat_body,
        out_type=(jax.ShapeDtypeStruct((2, NP, HW), jnp.float32),
                  jax.ShapeDtypeStruct((2, NPH, 16), jnp.float32)),
        mesh=plsc.VectorSubcoreMesh(core_axis_name="c", subcore_axis_name="s"),
        compiler_params=pltpu.CompilerParams(needs_layout_passes=False,
                                             use_tc_tiling_on_sc=False),
        scratch_types=[
            pltpu.VMEM((CPT, CH), jnp.int32),
            pltpu.VMEM((2, CH), jnp.int32),
            pltpu.VMEM((2, CH), jnp.int32),
            pltpu.VMEM((2, CH), jnp.int32),
            pltpu.VMEM((NP,), jnp.float32),
            pltpu.VMEM((NP,), jnp.float32),
            pltpu.VMEM((NP,), jnp.float32),
            pltpu.VMEM((2, CH, HW), jnp.bfloat16),
            pltpu.VMEM((2, CH, HW), jnp.float32),
            pltpu.VMEM((2, CH, 16), jnp.float32),
            pltpu.VMEM_SHARED((NP, HW), jnp.float32),
            pltpu.VMEM_SHARED((NPH, 16), jnp.float32),
            pltpu.VMEM_SHARED((NHP, HW), jnp.bfloat16),
            pltpu.SemaphoreType.DMA,
            pltpu.SemaphoreType.DMA,
        ],
    )
    return f(pk3d, asrc, adst, shift, hp2)


def _tc2_body(out_ref, den_ref, b_ref, w_ref, as_ref, ad_ref,
              h_ref, s_ref, d_ref):
    agg = jnp.concatenate([out_ref[0], out_ref[1]], axis=1)
    den = den_ref[...]
    x = _gelu(agg / (den + 1e-16) + b_ref[...])
    h = jnp.dot(x, w_ref[...], preferred_element_type=jnp.float32)
    h_ref[...] = h
    s_ref[...] = jnp.sum(h * as_ref[...], axis=1)
    d_ref[...] = jnp.sum(h * ad_ref[...], axis=1)


def _tc2(out1, den1, b, W, a_s, a_d):
    grid = ((N + _BLK - 1) // _BLK,)
    return pl.pallas_call(
        _tc2_body,
        grid=grid,
        in_specs=[
            pl.BlockSpec((2, _BLK, HW), lambda i: (0, i, 0)),
            pl.BlockSpec((_BLK, 1), lambda i: (i, 0)),
            pl.BlockSpec((1, Z_DIM), lambda i: (0, 0)),
            pl.BlockSpec((Z_DIM, Z_DIM), lambda i: (0, 0)),
            pl.BlockSpec((1, Z_DIM), lambda i: (0, 0)),
            pl.BlockSpec((1, Z_DIM), lambda i: (0, 0)),
        ],
        out_specs=[
            pl.BlockSpec((_BLK, Z_DIM), lambda i: (i, 0)),
            pl.BlockSpec((_BLK,), lambda i: (i,)),
            pl.BlockSpec((_BLK,), lambda i: (i,)),
        ],
        out_shape=[
            jax.ShapeDtypeStruct((N, Z_DIM), jnp.float32),
            jax.ShapeDtypeStruct((N,), jnp.float32),
            jax.ShapeDtypeStruct((N,), jnp.float32),
        ],
    )(out1, den1, b.reshape(1, Z_DIM), W,
      a_s.reshape(1, Z_DIM), a_d.reshape(1, Z_DIM))


def _tc3_body(out_ref, den_ref, b_ref, gcw_ref, gcb_ref, genw_ref, genb_ref,
              decw_ref, decb_ref, x_ref, acc_ref):
    i = pl.program_id(0)
    agg = jnp.concatenate([out_ref[0], out_ref[1]], axis=1)
    den = den_ref[...]
    z = _gelu(agg / (den + 1e-16) + b_ref[...])
    z = jnp.dot(z, gcw_ref[...], preferred_element_type=jnp.float32) + gcb_ref[...]
    z = jnp.dot(z, genw_ref[...], preferred_element_type=jnp.float32) + genb_ref[...]
    xh = jnp.dot(z, decw_ref[...], preferred_element_type=jnp.float32) + decb_ref[...]
    d = xh - x_ref[...]
    rows = i * _BLK + lax.broadcasted_iota(jnp.int32, (_BLK, IN_DIM), 0)
    d = jnp.where(rows < N, d, 0.0)
    s = jnp.sum(d * d).reshape(1, 1)

    @pl.when(i == 0)
    def _():
        acc_ref[...] = jnp.zeros((1, 1), jnp.float32)
    acc_ref[...] += s


def _tc3(out2, den2, b, gcW, gcb, genW, genb, decW, decb, X):
    grid = ((N + _BLK - 1) // _BLK,)
    return pl.pallas_call(
        _tc3_body,
        grid=grid,
        in_specs=[
            pl.BlockSpec((2, _BLK, HW), lambda i: (0, i, 0)),
            pl.BlockSpec((_BLK, 1), lambda i: (i, 0)),
            pl.BlockSpec((1, Z_DIM), lambda i: (0, 0)),
            pl.BlockSpec((Z_DIM, Z_DIM), lambda i: (0, 0)),
            pl.BlockSpec((1, Z_DIM), lambda i: (0, 0)),
            pl.BlockSpec((Z_DIM, Z_DIM), lambda i: (0, 0)),
            pl.BlockSpec((1, Z_DIM), lambda i: (0, 0)),
            pl.BlockSpec((Z_DIM, IN_DIM), lambda i: (0, 0)),
            pl.BlockSpec((1, IN_DIM), lambda i: (0, 0)),
            pl.BlockSpec((_BLK, IN_DIM), lambda i: (i, 0)),
        ],
        out_specs=pl.BlockSpec((1, 1), lambda i: (0, 0)),
        out_shape=jax.ShapeDtypeStruct((1, 1), jnp.float32),
    )(out2, den2, b.reshape(1, Z_DIM), gcW, gcb.reshape(1, Z_DIM),
      genW, genb.reshape(1, Z_DIM), decW, decb.reshape(1, IN_DIM), X)


def _shift_and_pad(a_s, a_d):
    t = jnp.max(a_s) + a_d
    shift = jnp.maximum(t, 0.2 * t)
    pad = NP - N
    return (jnp.pad(a_s, (0, pad)), jnp.pad(a_d, (0, pad)),
            jnp.pad(shift, (0, pad)))


def _split_cols(hp):
    def prep(hh):
        return jnp.stack([hh[:, :16], hh[:, 16:]], axis=2).reshape(NHP, HW)
    return jnp.stack([prep(hp[:, :HW]),
                      prep(hp[:, HW:])]).astype(jnp.bfloat16)


def _unpack_den(den):
    return den[0].reshape(NPH * 8, 2)[:, 0:1]


def kernel(X, edge_index, edge_weight, fn_W1, fn_b1, fn_W2, fn_b2,
           gat1_W, gat1_as, gat1_ad, gat1_b,
           gat2_W, gat2_as, gat2_ad, gat2_b,
           gc_W, gc_b, gen_W, gen_b, dec_W, dec_b):
    loop = jnp.arange(N, dtype=jnp.int32)
    padi = jnp.full((EP - EA,), N, jnp.int32)
    src_all = jnp.concatenate([edge_index[0], loop, padi])
    dst_all = jnp.concatenate([edge_index[1], loop, padi])
    pk3d = (src_all | (dst_all << 14)).reshape(NSUB, CPT, CH)
    pad = NP - N

    h1, as1, ad1 = _tc1(X, fn_W1, fn_b1, fn_W2, fn_b2, gat1_W, gat1_as, gat1_ad)
    asrc1, adst1, shift1 = _shift_and_pad(as1, ad1)
    hp1 = _split_cols(jnp.pad(h1, ((0, NHP - N), (0, 0))))
    out1, den1 = _sc_gat(pk3d, asrc1, adst1, shift1, hp1)

    h2, as2, ad2 = _tc2(out1, _unpack_den(den1), gat1_b, gat2_W, gat2_as, gat2_ad)
    asrc2, adst2, shift2 = _shift_and_pad(as2, ad2)
    hp2 = _split_cols(jnp.pad(h2, ((0, NHP - N), (0, 0))))
    out2, den2 = _sc_gat(pk3d, asrc2, adst2, shift2, hp2)

    acc = _tc3(out2, _unpack_den(den2), gat2_b, gc_W, gc_b, gen_W, gen_b,
               dec_W, dec_b, X)
    return acc[0, 0] / float(N * IN_DIM)


# speedup vs baseline: 1.3263x; 1.3263x over previous
import jax
import jax.numpy as jnp
from jax import lax
from jax.experimental import pallas as pl
from jax.experimental.pallas import tpu as pltpu
from jax.experimental.pallas import tpu_sc as plsc

N = 10000
IN_DIM = 128
H_DIM = 128
Z_DIM = 64
HW = Z_DIM // 2

NP = 10112
E = 320000
EA = E + N
CH = 128
NSUB = 16
CPT = 164
EP = NSUB * CPT * CH
RPT = NP // 16
NPH = 5120
RPTH = NPH // 16
NHP = 10016
RHPT = NHP // 16

_BLK = 1024


def _gelu(x):
    return 0.5 * x * (1.0 + jax.lax.erf(x * 0.7071067811865476))


def _tc1_body(x_ref, w1_ref, b1_ref, w2_ref, b2_ref, gw_ref, as_ref, ad_ref,
              h_ref, s_ref, d_ref):
    z = _gelu(jnp.dot(x_ref[...], w1_ref[...],
                      preferred_element_type=jnp.float32) + b1_ref[...])
    z = _gelu(jnp.dot(z, w2_ref[...],
                      preferred_element_type=jnp.float32) + b2_ref[...])
    h = jnp.dot(z, gw_ref[...], preferred_element_type=jnp.float32)
    h_ref[...] = h
    s_ref[...] = jnp.sum(h * as_ref[...], axis=1)
    d_ref[...] = jnp.sum(h * ad_ref[...], axis=1)


def _tc1(X, W1, b1, W2, b2, gW, a_s, a_d):
    grid = ((N + _BLK - 1) // _BLK,)
    return pl.pallas_call(
        _tc1_body,
        grid=grid,
        in_specs=[
            pl.BlockSpec((_BLK, IN_DIM), lambda i: (i, 0)),
            pl.BlockSpec((IN_DIM, H_DIM), lambda i: (0, 0)),
            pl.BlockSpec((H_DIM,), lambda i: (0,)),
            pl.BlockSpec((H_DIM, H_DIM), lambda i: (0, 0)),
            pl.BlockSpec((H_DIM,), lambda i: (0,)),
            pl.BlockSpec((H_DIM, Z_DIM), lambda i: (0, 0)),
            pl.BlockSpec((1, Z_DIM), lambda i: (0, 0)),
            pl.BlockSpec((1, Z_DIM), lambda i: (0, 0)),
        ],
        out_specs=[
            pl.BlockSpec((_BLK, Z_DIM), lambda i: (i, 0)),
            pl.BlockSpec((_BLK,), lambda i: (i,)),
            pl.BlockSpec((_BLK,), lambda i: (i,)),
        ],
        out_shape=[
            jax.ShapeDtypeStruct((N, Z_DIM), jnp.float32),
            jax.ShapeDtypeStruct((N,), jnp.float32),
            jax.ShapeDtypeStruct((N,), jnp.float32),
        ],
    )(X, W1, b1, W2, b2, gW, a_s.reshape(1, Z_DIM), a_d.reshape(1, Z_DIM))


def _sc_gat_body(pk_h, asrc_h, adst_h, shift_h, hp_h,
                 out_h, den_h,
                 pk_v, srcr, dstr, dst2r, asrc_v, adst_v, shift_v,
                 hbrows, frows, erows,
                 out_sp, den_sp, hp_sp, g_sem, s_sem):
    cid = lax.axis_index("c")
    sid = lax.axis_index("s")
    rowbase = sid * RPT

    pltpu.sync_copy(asrc_h, asrc_v)
    pltpu.sync_copy(adst_h, adst_v)
    pltpu.sync_copy(shift_h, shift_v)
    pltpu.sync_copy(pk_h.at[sid], pk_v)
    pltpu.sync_copy(hp_h.at[cid].at[pl.ds(sid * RHPT, RHPT)],
                    hp_sp.at[pl.ds(sid * RHPT, RHPT)])

    def zrow(r, carry):
        for b in range(2):
            for g in range(HW // 16):
                frows[b, r, pl.ds(g * 16, 16)] = jnp.zeros((16,), jnp.float32)
            erows[b, r, :] = jnp.zeros((16,), jnp.float32)
        return carry
    lax.fori_loop(0, CH, zrow, 0)

    def zcopy(k, carry):
        pltpu.sync_copy(frows.at[0].at[pl.ds(0, 128)],
                        out_sp.at[pl.ds(rowbase + k * 128, 128)])
        return carry
    lax.fori_loop(0, 4, zcopy, 0)
    pltpu.sync_copy(frows.at[0].at[pl.ds(0, 120)],
                    out_sp.at[pl.ds(rowbase + 512, 120)])
    rowbase_h = sid * RPTH
    def zden(k, carry):
        pltpu.sync_copy(erows.at[0].at[pl.ds(0, 64)],
                        den_sp.at[pl.ds(rowbase_h + k * 64, 64)])
        return carry
    lax.fori_loop(0, RPTH // 64, zden, 0)

    plsc.subcore_barrier()

    zero16 = jnp.zeros((16,), jnp.int32)
    lane = lax.iota(jnp.int32, 16)

    def decode(ck, b):
        def d_body(j, c2):
            sl = pl.ds(j * 16, 16)
            p = pk_v[ck, sl]
            d16 = lax.shift_right_logical(p, 14)
            srcr[b, sl] = p & 16383
            dstr[b, sl] = d16
            dst2r[b, sl] = lax.shift_right_logical(d16, 1)
            return c2
        lax.fori_loop(0, CH // 16, d_body, 0)

    def e_compute(b, er_ref):
        def e_body(j, c2):
            s16 = srcr[b, pl.ds(j * 16, 16)]
            d16 = dstr[b, pl.ds(j * 16, 16)]
            a = plsc.load_gather(asrc_v, [s16]) + plsc.load_gather(adst_v, [d16])
            a = jnp.maximum(a, 0.2 * a)
            e16 = jnp.exp(a - plsc.load_gather(shift_v, [d16]))
            p8 = (d16 & 1) * 8
            row = j * 16 + lane
            plsc.store_scatter(er_ref, [row, 8 - p8],
                               jnp.zeros((16,), jnp.float32))
            plsc.store_scatter(er_ref, [row, p8], e16)
            return c2
        lax.fori_loop(0, CH // 16, e_body, 0)

    def mult(b, er_ref):
        def m_body(i, c2):
            ers = []
            unpacked = []
            for k in range(4):
                r = 4 * i + k
                ers.append(plsc.load_gather(
                    er_ref, [jnp.full((16,), r, jnp.int32), zero16]))
                unpacked.append(plsc.unpack(
                    hbrows[b, r, :], format=plsc.PackFormat.INTERLEAVED))
            for k in range(4):
                r = 4 * i + k
                ea, eb = unpacked[k]
                frows[b, r, pl.ds(0, 16)] = ea * ers[k]
                frows[b, r, pl.ds(16, 16)] = eb * ers[k]
            return c2
        lax.fori_loop(0, CH // 4, m_body, 0)

    def issue_gather(b):
        pltpu.async_copy(hp_sp.at[srcr.at[b]], hbrows.at[b], g_sem)

    def wait_gather(b):
        pltpu.make_async_copy(hp_sp.at[srcr.at[b]], hbrows.at[b],
                              g_sem).wait()

    def issue_scatter(b):
        pltpu.async_copy(frows.at[b], out_sp.at[dstr.at[b]], s_sem, add=True)
        pltpu.async_copy(erows.at[b], den_sp.at[dst2r.at[b]], s_sem,
                         add=True)

    def wait_scatter(b):
        pltpu.make_async_copy(frows.at[b], out_sp.at[dstr.at[b]],
                              s_sem).wait()
        pltpu.make_async_copy(erows.at[b], den_sp.at[dst2r.at[b]],
                              s_sem).wait()

    decode(0, 0)
    issue_gather(0)

    def chunk_body(ck, carry):
        b = ck & 1

        @pl.when(ck > 0)
        def _():
            wait_scatter(1 - b)

        @pl.when(ck + 1 < CPT)
        def _():
            decode(ck + 1, 1 - b)
            issue_gather(1 - b)
        e_compute(b, erows.at[b])
        wait_gather(b)
        mult(b, erows.at[b])
        issue_scatter(b)
        return carry
    lax.fori_loop(0, CPT, chunk_body, 0)
    wait_scatter(1)

    plsc.subcore_barrier()
    pltpu.sync_copy(out_sp.at[pl.ds(rowbase, RPT)],
                    out_h.at[cid].at[pl.ds(rowbase, RPT)])
    pltpu.sync_copy(den_sp.at[pl.ds(rowbase_h, RPTH)],
                    den_h.at[cid].at[pl.ds(rowbase_h, RPTH)])


def _sc_gat(pk3d, asrc, adst, shift, hp2):
    f = pl.kernel(
        _sc_gat_body,
        out_type=(jax.ShapeDtypeStruct((2, NP, HW), jnp.float32),
                  jax.ShapeDtypeStruct((2, NPH, 16), jnp.float32)),
        mesh=plsc.VectorSubcoreMesh(core_axis_name="c", subcore_axis_name="s"),
        compiler_params=pltpu.CompilerParams(needs_layout_passes=False,
                                             use_tc_tiling_on_sc=False),
        scratch_types=[
            pltpu.VMEM((CPT, CH), jnp.int32),
            pltpu.VMEM((2, CH), jnp.int32),
            pltpu.VMEM((2, CH), jnp.int32),
            pltpu.VMEM((2, CH), jnp.int32),
            pltpu.VMEM((NP,), jnp.float32),
            pltpu.VMEM((NP,), jnp.float32),
            pltpu.VMEM((NP,), jnp.float32),
            pltpu.VMEM((2, CH, HW), jnp.bfloat16),
            pltpu.VMEM((2, CH, HW), jnp.float32),
            pltpu.VMEM((2, CH, 16), jnp.float32),
            pltpu.VMEM_SHARED((NP, HW), jnp.float32),
            pltpu.VMEM_SHARED((NPH, 16), jnp.float32),
            pltpu.VMEM_SHARED((NHP, HW), jnp.bfloat16),
            pltpu.SemaphoreType.DMA,
            pltpu.SemaphoreType.DMA,
        ],
    )
    return f(pk3d, asrc, adst, shift, hp2)


def _tc2_body(out_ref, den_ref, b_ref, w_ref, as_ref, ad_ref,
              h_ref, s_ref, d_ref):
    agg = jnp.concatenate([out_ref[0], out_ref[1]], axis=1)
    den = den_ref[...]
    x = _gelu(agg / (den + 1e-16) + b_ref[...])
    h = jnp.dot(x, w_ref[...], preferred_element_type=jnp.float32)
    h_ref[...] = h
    s_ref[...] = jnp.sum(h * as_ref[...], axis=1)
    d_ref[...] = jnp.sum(h * ad_ref[...], axis=1)


def _tc2(out1, den1, b, W, a_s, a_d):
    grid = ((N + _BLK - 1) // _BLK,)
    return pl.pallas_call(
        _tc2_body,
        grid=grid,
        in_specs=[
            pl.BlockSpec((2, _BLK, HW), lambda i: (0, i, 0)),
            pl.BlockSpec((_BLK, 1), lambda i: (i, 0)),
            pl.BlockSpec((1, Z_DIM), lambda i: (0, 0)),
            pl.BlockSpec((Z_DIM, Z_DIM), lambda i: (0, 0)),
            pl.BlockSpec((1, Z_DIM), lambda i: (0, 0)),
            pl.BlockSpec((1, Z_DIM), lambda i: (0, 0)),
        ],
        out_specs=[
            pl.BlockSpec((_BLK, Z_DIM), lambda i: (i, 0)),
            pl.BlockSpec((_BLK,), lambda i: (i,)),
            pl.BlockSpec((_BLK,), lambda i: (i,)),
        ],
        out_shape=[
            jax.ShapeDtypeStruct((N, Z_DIM), jnp.float32),
            jax.ShapeDtypeStruct((N,), jnp.float32),
            jax.ShapeDtypeStruct((N,), jnp.float32),
        ],
    )(out1, den1, b.reshape(1, Z_DIM), W,
      a_s.reshape(1, Z_DIM), a_d.reshape(1, Z_DIM))


def _tc3_body(out_ref, den_ref, b_ref, gcw_ref, gcb_ref, genw_ref, genb_ref,
              decw_ref, decb_ref, x_ref, acc_ref):
    i = pl.program_id(0)
    agg = jnp.concatenate([out_ref[0], out_ref[1]], axis=1)
    den = den_ref[...]
    z = _gelu(agg / (den + 1e-16) + b_ref[...])
    z = jnp.dot(z, gcw_ref[...], preferred_element_type=jnp.float32) + gcb_ref[...]
    z = jnp.dot(z, genw_ref[...], preferred_element_type=jnp.float32) + genb_ref[...]
    xh = jnp.dot(z, decw_ref[...], preferred_element_type=jnp.float32) + decb_ref[...]
    d = xh - x_ref[...]
    rows = i * _BLK + lax.broadcasted_iota(jnp.int32, (_BLK, IN_DIM), 0)
    d = jnp.where(rows < N, d, 0.0)
    s = jnp.sum(d * d).reshape(1, 1)

    @pl.when(i == 0)
    def _():
        acc_ref[...] = jnp.zeros((1, 1), jnp.float32)
    acc_ref[...] += s


def _tc3(out2, den2, b, gcW, gcb, genW, genb, decW, decb, X):
    grid = ((N + _BLK - 1) // _BLK,)
    return pl.pallas_call(
        _tc3_body,
        grid=grid,
        in_specs=[
            pl.BlockSpec((2, _BLK, HW), lambda i: (0, i, 0)),
            pl.BlockSpec((_BLK, 1), lambda i: (i, 0)),
            pl.BlockSpec((1, Z_DIM), lambda i: (0, 0)),
            pl.BlockSpec((Z_DIM, Z_DIM), lambda i: (0, 0)),
            pl.BlockSpec((1, Z_DIM), lambda i: (0, 0)),
            pl.BlockSpec((Z_DIM, Z_DIM), lambda i: (0, 0)),
            pl.BlockSpec((1, Z_DIM), lambda i: (0, 0)),
            pl.BlockSpec((Z_DIM, IN_DIM), lambda i: (0, 0)),
            pl.BlockSpec((1, IN_DIM), lambda i: (0, 0)),
            pl.BlockSpec((_BLK, IN_DIM), lambda i: (i, 0)),
        ],
        out_specs=pl.BlockSpec((1, 1), lambda i: (0, 0)),
        out_shape=jax.ShapeDtypeStruct((1, 1), jnp.float32),
    )(out2, den2, b.reshape(1, Z_DIM), gcW, gcb.reshape(1, Z_DIM),
      genW, genb.reshape(1, Z_DIM), decW, decb.reshape(1, IN_DIM), X)


def _shift_and_pad(a_s, a_d):
    t = jnp.max(a_s) + a_d
    shift = jnp.maximum(t, 0.2 * t)
    pad = NP - N
    return (jnp.pad(a_s, (0, pad)), jnp.pad(a_d, (0, pad)),
            jnp.pad(shift, (0, pad)))


def _split_cols(hp):
    def prep(hh):
        return jnp.stack([hh[:, :16], hh[:, 16:]], axis=2).reshape(NHP, HW)
    return jnp.stack([prep(hp[:, :HW]),
                      prep(hp[:, HW:])]).astype(jnp.bfloat16)


def _unpack_den(den):
    return den[0].reshape(NPH * 2, 8)[:, 0:1]


def kernel(X, edge_index, edge_weight, fn_W1, fn_b1, fn_W2, fn_b2,
           gat1_W, gat1_as, gat1_ad, gat1_b,
           gat2_W, gat2_as, gat2_ad, gat2_b,
           gc_W, gc_b, gen_W, gen_b, dec_W, dec_b):
    loop = jnp.arange(N, dtype=jnp.int32)
    padi = jnp.full((EP - EA,), N, jnp.int32)
    src_all = jnp.concatenate([edge_index[0], loop, padi])
    dst_all = jnp.concatenate([edge_index[1], loop, padi])
    pk3d = (src_all | (dst_all << 14)).reshape(NSUB, CPT, CH)
    pad = NP - N

    h1, as1, ad1 = _tc1(X, fn_W1, fn_b1, fn_W2, fn_b2, gat1_W, gat1_as, gat1_ad)
    asrc1, adst1, shift1 = _shift_and_pad(as1, ad1)
    hp1 = _split_cols(jnp.pad(h1, ((0, NHP - N), (0, 0))))
    out1, den1 = _sc_gat(pk3d, asrc1, adst1, shift1, hp1)

    h2, as2, ad2 = _tc2(out1, _unpack_den(den1), gat1_b, gat2_W, gat2_as, gat2_ad)
    asrc2, adst2, shift2 = _shift_and_pad(as2, ad2)
    hp2 = _split_cols(jnp.pad(h2, ((0, NHP - N), (0, 0))))
    out2, den2 = _sc_gat(pk3d, asrc2, adst2, shift2, hp2)

    acc = _tc3(out2, _unpack_den(den2), gat2_b, gc_W, gc_b, gen_W, gen_b,
               dec_W, dec_b, X)
    return acc[0, 0] / float(N * IN_DIM)


# speedup vs baseline: 1.5544x; 1.1720x over previous
import jax
import jax.numpy as jnp
from jax import lax
from jax.experimental import pallas as pl
from jax.experimental.pallas import tpu as pltpu
from jax.experimental.pallas import tpu_sc as plsc

N = 10000
IN_DIM = 128
H_DIM = 128
Z_DIM = 64
HW = Z_DIM // 2

NP = 10016
E = 320000
EA = E + N
CH = 128
NWORK = 32
CPT = 81
EP = NWORK * CPT * CH
RPT = NP // 16
NPH = 1264
RPTH = NPH // 16
NHP = 10016
RHPT = NHP // 16

_BLK = 1024


def _gelu(x):
    return 0.5 * x * (1.0 + jax.lax.erf(x * 0.7071067811865476))


def _tc1_body(x_ref, w1_ref, b1_ref, w2_ref, b2_ref, gw_ref, as_ref, ad_ref,
              h_ref, s_ref, d_ref):
    z = _gelu(jnp.dot(x_ref[...], w1_ref[...],
                      preferred_element_type=jnp.float32) + b1_ref[...])
    z = _gelu(jnp.dot(z, w2_ref[...],
                      preferred_element_type=jnp.float32) + b2_ref[...])
    h = jnp.dot(z, gw_ref[...], preferred_element_type=jnp.float32)
    h_ref[...] = h
    s_ref[...] = jnp.sum(h * as_ref[...], axis=1)
    d_ref[...] = jnp.sum(h * ad_ref[...], axis=1)


def _tc1(X, W1, b1, W2, b2, gW, a_s, a_d):
    grid = ((N + _BLK - 1) // _BLK,)
    return pl.pallas_call(
        _tc1_body,
        grid=grid,
        in_specs=[
            pl.BlockSpec((_BLK, IN_DIM), lambda i: (i, 0)),
            pl.BlockSpec((IN_DIM, H_DIM), lambda i: (0, 0)),
            pl.BlockSpec((H_DIM,), lambda i: (0,)),
            pl.BlockSpec((H_DIM, H_DIM), lambda i: (0, 0)),
            pl.BlockSpec((H_DIM,), lambda i: (0,)),
            pl.BlockSpec((H_DIM, Z_DIM), lambda i: (0, 0)),
            pl.BlockSpec((1, Z_DIM), lambda i: (0, 0)),
            pl.BlockSpec((1, Z_DIM), lambda i: (0, 0)),
        ],
        out_specs=[
            pl.BlockSpec((_BLK, Z_DIM), lambda i: (i, 0)),
            pl.BlockSpec((_BLK,), lambda i: (i,)),
            pl.BlockSpec((_BLK,), lambda i: (i,)),
        ],
        out_shape=[
            jax.ShapeDtypeStruct((N, Z_DIM), jnp.float32),
            jax.ShapeDtypeStruct((N,), jnp.float32),
            jax.ShapeDtypeStruct((N,), jnp.float32),
        ],
    )(X, W1, b1, W2, b2, gW, a_s.reshape(1, Z_DIM), a_d.reshape(1, Z_DIM))


def _sc_gat_body(pk_h, asrc_h, adst_h, hp_h,
                 out_h, den_h,
                 pk_v, srcr, dstr, dst2r, asrc_v, adst_v,
                 hbrows, frows, erows,
                 out_sp, den_sp, hp_sp, g_sem, s_sem):
    cid = lax.axis_index("c")
    sid = lax.axis_index("s")
    rowbase = sid * RPT

    pltpu.sync_copy(asrc_h, asrc_v)
    pltpu.sync_copy(adst_h, adst_v)
    pltpu.sync_copy(pk_h.at[cid * 16 + sid], pk_v)
    maxas = plsc.load_gather(asrc_v, [jnp.full((16,), NP - 1, jnp.int32)])
    pltpu.sync_copy(hp_h.at[pl.ds(sid * RHPT, RHPT)],
                    hp_sp.at[pl.ds(sid * RHPT, RHPT)])

    def zrow(r, carry):
        for b in range(2):
            for g in range(Z_DIM // 16):
                frows[b, r, pl.ds(g * 16, 16)] = jnp.zeros((16,), jnp.float32)
            erows[b, r, :] = jnp.zeros((16,), jnp.float32)
        return carry
    lax.fori_loop(0, CH, zrow, 0)

    def zcopy(k, carry):
        pltpu.sync_copy(frows.at[0].at[pl.ds(0, 128)],
                        out_sp.at[pl.ds(rowbase + k * 128, 128)])
        return carry
    lax.fori_loop(0, 4, zcopy, 0)
    pltpu.sync_copy(frows.at[0].at[pl.ds(0, 114)],
                    out_sp.at[pl.ds(rowbase + 512, 114)])
    rowbase_h = sid * RPTH
    pltpu.sync_copy(erows.at[0].at[pl.ds(0, RPTH)],
                    den_sp.at[pl.ds(rowbase_h, RPTH)])

    plsc.subcore_barrier()

    zero16 = jnp.zeros((16,), jnp.int32)
    lane = lax.iota(jnp.int32, 16)

    def decode(ck, b):
        def d_body(j, c2):
            sl = pl.ds(j * 16, 16)
            p = pk_v[ck, sl]
            d16 = lax.shift_right_logical(p, 14)
            srcr[b, sl] = p & 16383
            dstr[b, sl] = d16
            dst2r[b, sl] = lax.shift_right_logical(d16, 3)
            return c2
        lax.fori_loop(0, CH // 16, d_body, 0)

    def e_compute(b, er_ref):
        def e_body(j, c2):
            s16 = srcr[b, pl.ds(j * 16, 16)]
            d16 = dstr[b, pl.ds(j * 16, 16)]
            dg = plsc.load_gather(adst_v, [d16])
            a = plsc.load_gather(asrc_v, [s16]) + dg
            a = jnp.maximum(a, 0.2 * a)
            t = maxas + dg
            e16 = jnp.exp(a - jnp.maximum(t, 0.2 * t))
            p2 = (d16 & 7) * 2
            zf = jnp.zeros((16,), jnp.float32)
            row = j * 16 + lane
            for q in range(2, 16, 2):
                plsc.store_scatter(er_ref, [row, (p2 + q) & 14], zf)
            plsc.store_scatter(er_ref, [row, p2], e16)
            return c2
        lax.fori_loop(0, CH // 16, e_body, 0)

    def mult(b, er_ref):
        def m_body(i, c2):
            ers = []
            unpacked = []
            for k in range(2):
                r = 2 * i + k
                ers.append(plsc.load_gather(
                    er_ref, [jnp.full((16,), r, jnp.int32), zero16]))
                for g in range(2):
                    unpacked.append(plsc.unpack(
                        hbrows[b, r, pl.ds(g * 32, 32)],
                        format=plsc.PackFormat.INTERLEAVED))
            for k in range(2):
                r = 2 * i + k
                for g in range(2):
                    ea, eb = unpacked[2 * k + g]
                    frows[b, r, pl.ds(g * 32, 16)] = ea * ers[k]
                    frows[b, r, pl.ds(g * 32 + 16, 16)] = eb * ers[k]
            return c2
        lax.fori_loop(0, CH // 2, m_body, 0)

    def issue_gather(b):
        pltpu.async_copy(hp_sp.at[srcr.at[b]], hbrows.at[b], g_sem)

    def wait_gather(b):
        pltpu.make_async_copy(hp_sp.at[srcr.at[b]], hbrows.at[b],
                              g_sem).wait()

    def issue_scatter(b):
        pltpu.async_copy(frows.at[b], out_sp.at[dstr.at[b]], s_sem, add=True)
        pltpu.async_copy(erows.at[b], den_sp.at[dst2r.at[b]], s_sem,
                         add=True)

    def wait_scatter(b):
        pltpu.make_async_copy(frows.at[b], out_sp.at[dstr.at[b]],
                              s_sem).wait()
        pltpu.make_async_copy(erows.at[b], den_sp.at[dst2r.at[b]],
                              s_sem).wait()

    decode(0, 0)
    issue_gather(0)

    def chunk_body(ck, carry):
        b = ck & 1

        @pl.when(ck > 0)
        def _():
            wait_scatter(1 - b)

        @pl.when(ck + 1 < CPT)
        def _():
            decode(ck + 1, 1 - b)
            issue_gather(1 - b)
        e_compute(b, erows.at[b])
        wait_gather(b)
        mult(b, erows.at[b])
        issue_scatter(b)
        return carry
    lax.fori_loop(0, CPT, chunk_body, 0)
    wait_scatter((CPT - 1) & 1)

    plsc.subcore_barrier()
    pltpu.sync_copy(out_sp.at[pl.ds(rowbase, RPT)],
                    out_h.at[cid].at[pl.ds(rowbase, RPT)])
    pltpu.sync_copy(den_sp.at[pl.ds(rowbase_h, RPTH)],
                    den_h.at[cid].at[pl.ds(rowbase_h, RPTH)])


def _sc_gat(pk3d, asrc, adst, hp2):
    f = pl.kernel(
        _sc_gat_body,
        out_type=(jax.ShapeDtypeStruct((2, NP, Z_DIM), jnp.float32),
                  jax.ShapeDtypeStruct((2, NPH, 16), jnp.float32)),
        mesh=plsc.VectorSubcoreMesh(core_axis_name="c", subcore_axis_name="s"),
        compiler_params=pltpu.CompilerParams(needs_layout_passes=False,
                                             use_tc_tiling_on_sc=False),
        scratch_types=[
            pltpu.VMEM((CPT, CH), jnp.int32),
            pltpu.VMEM((2, CH), jnp.int32),
            pltpu.VMEM((2, CH), jnp.int32),
            pltpu.VMEM((2, CH), jnp.int32),
            pltpu.VMEM((NP,), jnp.float32),
            pltpu.VMEM((NP,), jnp.float32),
            pltpu.VMEM((2, CH, Z_DIM), jnp.bfloat16),
            pltpu.VMEM((2, CH, Z_DIM), jnp.float32),
            pltpu.VMEM((2, CH, 16), jnp.float32),
            pltpu.VMEM_SHARED((NP, Z_DIM), jnp.float32),
            pltpu.VMEM_SHARED((NPH, 16), jnp.float32),
            pltpu.VMEM_SHARED((NHP, Z_DIM), jnp.bfloat16),
            pltpu.SemaphoreType.DMA,
            pltpu.SemaphoreType.DMA,
        ],
    )
    return f(pk3d, asrc, adst, hp2)


def _tc2_body(out_ref, den_ref, b_ref, w_ref, as_ref, ad_ref,
              h_ref, s_ref, d_ref):
    agg = out_ref[0] + out_ref[1]
    den = den_ref[...]
    x = _gelu(agg / (den + 1e-16) + b_ref[...])
    h = jnp.dot(x, w_ref[...], preferred_element_type=jnp.float32)
    h_ref[...] = h
    s_ref[...] = jnp.sum(h * as_ref[...], axis=1)
    d_ref[...] = jnp.sum(h * ad_ref[...], axis=1)


def _tc2(out1, den1, b, W, a_s, a_d):
    grid = ((N + _BLK - 1) // _BLK,)
    return pl.pallas_call(
        _tc2_body,
        grid=grid,
        in_specs=[
            pl.BlockSpec((2, _BLK, Z_DIM), lambda i: (0, i, 0)),
            pl.BlockSpec((_BLK, 1), lambda i: (i, 0)),
            pl.BlockSpec((1, Z_DIM), lambda i: (0, 0)),
            pl.BlockSpec((Z_DIM, Z_DIM), lambda i: (0, 0)),
            pl.BlockSpec((1, Z_DIM), lambda i: (0, 0)),
            pl.BlockSpec((1, Z_DIM), lambda i: (0, 0)),
        ],
        out_specs=[
            pl.BlockSpec((_BLK, Z_DIM), lambda i: (i, 0)),
            pl.BlockSpec((_BLK,), lambda i: (i,)),
            pl.BlockSpec((_BLK,), lambda i: (i,)),
        ],
        out_shape=[
            jax.ShapeDtypeStruct((N, Z_DIM), jnp.float32),
            jax.ShapeDtypeStruct((N,), jnp.float32),
            jax.ShapeDtypeStruct((N,), jnp.float32),
        ],
    )(out1, den1, b.reshape(1, Z_DIM), W,
      a_s.reshape(1, Z_DIM), a_d.reshape(1, Z_DIM))


def _tc3_body(out_ref, den_ref, b_ref, gcw_ref, gcb_ref, genw_ref, genb_ref,
              decw_ref, decb_ref, x_ref, acc_ref):
    i = pl.program_id(0)
    agg = out_ref[0] + out_ref[1]
    den = den_ref[...]
    z = _gelu(agg / (den + 1e-16) + b_ref[...])
    z = jnp.dot(z, gcw_ref[...], preferred_element_type=jnp.float32) + gcb_ref[...]
    z = jnp.dot(z, genw_ref[...], preferred_element_type=jnp.float32) + genb_ref[...]
    xh = jnp.dot(z, decw_ref[...], preferred_element_type=jnp.float32) + decb_ref[...]
    d = xh - x_ref[...]
    rows = i * _BLK + lax.broadcasted_iota(jnp.int32, (_BLK, IN_DIM), 0)
    d = jnp.where(rows < N, d, 0.0)
    s = jnp.sum(d * d).reshape(1, 1)

    @pl.when(i == 0)
    def _():
        acc_ref[...] = jnp.zeros((1, 1), jnp.float32)
    acc_ref[...] += s


def _tc3(out2, den2, b, gcW, gcb, genW, genb, decW, decb, X):
    grid = ((N + _BLK - 1) // _BLK,)
    return pl.pallas_call(
        _tc3_body,
        grid=grid,
        in_specs=[
            pl.BlockSpec((2, _BLK, Z_DIM), lambda i: (0, i, 0)),
            pl.BlockSpec((_BLK, 1), lambda i: (i, 0)),
            pl.BlockSpec((1, Z_DIM), lambda i: (0, 0)),
            pl.BlockSpec((Z_DIM, Z_DIM), lambda i: (0, 0)),
            pl.BlockSpec((1, Z_DIM), lambda i: (0, 0)),
            pl.BlockSpec((Z_DIM, Z_DIM), lambda i: (0, 0)),
            pl.BlockSpec((1, Z_DIM), lambda i: (0, 0)),
            pl.BlockSpec((Z_DIM, IN_DIM), lambda i: (0, 0)),
            pl.BlockSpec((1, IN_DIM), lambda i: (0, 0)),
            pl.BlockSpec((_BLK, IN_DIM), lambda i: (i, 0)),
        ],
        out_specs=pl.BlockSpec((1, 1), lambda i: (0, 0)),
        out_shape=jax.ShapeDtypeStruct((1, 1), jnp.float32),
    )(out2, den2, b.reshape(1, Z_DIM), gcW, gcb.reshape(1, Z_DIM),
      genW, genb.reshape(1, Z_DIM), decW, decb.reshape(1, IN_DIM), X)


def _shift_and_pad(a_s, a_d):
    pad = NP - N
    asrc = jnp.pad(a_s, (0, pad)).at[NP - 1].set(jnp.max(a_s))
    return asrc, jnp.pad(a_d, (0, pad))


def _prep_h(hp):
    def prep(hh):
        return jnp.stack([hh[:, :16], hh[:, 16:]], axis=2).reshape(NHP, HW)
    return jnp.concatenate([prep(hp[:, :HW]), prep(hp[:, HW:])],
                           axis=1).astype(jnp.bfloat16)


def _unpack_den(den):
    return (den[0] + den[1]).reshape(NPH * 8, 2)[:, 0:1]


def kernel(X, edge_index, edge_weight, fn_W1, fn_b1, fn_W2, fn_b2,
           gat1_W, gat1_as, gat1_ad, gat1_b,
           gat2_W, gat2_as, gat2_ad, gat2_b,
           gc_W, gc_b, gen_W, gen_b, dec_W, dec_b):
    loop = jnp.arange(N, dtype=jnp.int32)
    padi = jnp.full((EP - EA,), N, jnp.int32)
    src_all = jnp.concatenate([edge_index[0], loop, padi])
    dst_all = jnp.concatenate([edge_index[1], loop, padi])
    pk3d = (src_all | (dst_all << 14)).reshape(NWORK, CPT, CH)
    pad = NP - N

    h1, as1, ad1 = _tc1(X, fn_W1, fn_b1, fn_W2, fn_b2, gat1_W, gat1_as, gat1_ad)
    asrc1, adst1 = _shift_and_pad(as1, ad1)
    hp1 = _prep_h(jnp.pad(h1, ((0, NHP - N), (0, 0))))
    out1, den1 = _sc_gat(pk3d, asrc1, adst1, hp1)

    h2, as2, ad2 = _tc2(out1, _unpack_den(den1), gat1_b, gat2_W, gat2_as, gat2_ad)
    asrc2, adst2 = _shift_and_pad(as2, ad2)
    hp2 = _prep_h(jnp.pad(h2, ((0, NHP - N), (0, 0))))
    out2, den2 = _sc_gat(pk3d, asrc2, adst2, hp2)

    acc = _tc3(out2, _unpack_den(den2), gat2_b, gc_W, gc_b, gen_W, gen_b,
               dec_W, dec_b, X)
    return acc[0, 0] / float(N * IN_DIM)


# speedup vs baseline: 1.8066x; 1.1623x over previous
import jax
import jax.numpy as jnp
from jax import lax
from jax.experimental import pallas as pl
from jax.experimental.pallas import tpu as pltpu
from jax.experimental.pallas import tpu_sc as plsc

N = 10000
IN_DIM = 128
H_DIM = 128
Z_DIM = 64
HW = Z_DIM // 2

NP = 10016
E = 320000
EA = E + N
CH = 128
NWORK = 32
CPT = 81
EP = NWORK * CPT * CH
RPT = NP // 16
NPH = 1264
RPTH = NPH // 16
NHP = 10016
RHPT = NHP // 16

_BLK = 1024


def _gelu(x):
    return 0.5 * x * (1.0 + jax.lax.erf(x * 0.7071067811865476))


def _tc1_body(x_ref, w1_ref, b1_ref, w2_ref, b2_ref, gw_ref, as_ref, ad_ref,
              h_ref, s_ref, d_ref):
    z = _gelu(jnp.dot(x_ref[...], w1_ref[...],
                      preferred_element_type=jnp.float32) + b1_ref[...])
    z = _gelu(jnp.dot(z, w2_ref[...],
                      preferred_element_type=jnp.float32) + b2_ref[...])
    h = jnp.dot(z, gw_ref[...], preferred_element_type=jnp.float32)
    h_ref[...] = h.astype(jnp.bfloat16)
    s_ref[...] = jnp.sum(h * as_ref[...], axis=1)
    d_ref[...] = jnp.sum(h * ad_ref[...], axis=1)


def _tc1(X, W1, b1, W2, b2, gW, a_s, a_d):
    grid = ((N + _BLK - 1) // _BLK,)
    return pl.pallas_call(
        _tc1_body,
        grid=grid,
        in_specs=[
            pl.BlockSpec((_BLK, IN_DIM), lambda i: (i, 0)),
            pl.BlockSpec((IN_DIM, H_DIM), lambda i: (0, 0)),
            pl.BlockSpec((H_DIM,), lambda i: (0,)),
            pl.BlockSpec((H_DIM, H_DIM), lambda i: (0, 0)),
            pl.BlockSpec((H_DIM,), lambda i: (0,)),
            pl.BlockSpec((H_DIM, Z_DIM), lambda i: (0, 0)),
            pl.BlockSpec((1, Z_DIM), lambda i: (0, 0)),
            pl.BlockSpec((1, Z_DIM), lambda i: (0, 0)),
        ],
        out_specs=[
            pl.BlockSpec((_BLK, Z_DIM), lambda i: (i, 0)),
            pl.BlockSpec((_BLK,), lambda i: (i,)),
            pl.BlockSpec((_BLK,), lambda i: (i,)),
        ],
        out_shape=[
            jax.ShapeDtypeStruct((N, Z_DIM), jnp.bfloat16),
            jax.ShapeDtypeStruct((N,), jnp.float32),
            jax.ShapeDtypeStruct((N,), jnp.float32),
        ],
    )(X, W1, b1, W2, b2, gW, a_s.reshape(1, Z_DIM), a_d.reshape(1, Z_DIM))


def _sc_gat_body(pk_h, asrc_h, adst_h, hp_h,
                 out_h, den_h,
                 pk_v, srcr, dstr, dst2r, asrc_v, adst_v,
                 hbrows, frows, erows,
                 out_sp, den_sp, hp_sp, g_sem, s_sem):
    cid = lax.axis_index("c")
    sid = lax.axis_index("s")
    rowbase = sid * RPT

    pltpu.sync_copy(asrc_h, asrc_v)
    pltpu.sync_copy(adst_h, adst_v)
    pltpu.sync_copy(pk_h.at[cid * 16 + sid], pk_v)
    maxas = plsc.load_gather(asrc_v, [jnp.full((16,), NP - 1, jnp.int32)])
    pltpu.sync_copy(hp_h.at[pl.ds(sid * RHPT, RHPT)],
                    hp_sp.at[pl.ds(sid * RHPT, RHPT)])

    def zrow(r, carry):
        for b in range(2):
            for g in range(Z_DIM // 16):
                frows[b, r, pl.ds(g * 16, 16)] = jnp.zeros((16,), jnp.float32)
            erows[b, r, :] = jnp.zeros((16,), jnp.float32)
        return carry
    lax.fori_loop(0, CH, zrow, 0)

    def zcopy(k, carry):
        pltpu.sync_copy(frows.at[0].at[pl.ds(0, 128)],
                        out_sp.at[pl.ds(rowbase + k * 128, 128)])
        return carry
    lax.fori_loop(0, 4, zcopy, 0)
    pltpu.sync_copy(frows.at[0].at[pl.ds(0, 114)],
                    out_sp.at[pl.ds(rowbase + 512, 114)])
    rowbase_h = sid * RPTH
    pltpu.sync_copy(erows.at[0].at[pl.ds(0, RPTH)],
                    den_sp.at[pl.ds(rowbase_h, RPTH)])

    plsc.subcore_barrier()

    zero16 = jnp.zeros((16,), jnp.int32)
    lane = lax.iota(jnp.int32, 16)

    def decode(ck, b):
        def d_body(j, c2):
            sl = pl.ds(j * 16, 16)
            p = pk_v[ck, sl]
            d16 = lax.shift_right_logical(p, 14)
            srcr[b, sl] = p & 16383
            dstr[b, sl] = d16
            dst2r[b, sl] = lax.shift_right_logical(d16, 3)
            return c2
        lax.fori_loop(0, CH // 16, d_body, 0)

    def e_compute(b, er_ref):
        def e_body(j, c2):
            s16 = srcr[b, pl.ds(j * 16, 16)]
            d16 = dstr[b, pl.ds(j * 16, 16)]
            dg = plsc.load_gather(adst_v, [d16])
            a = plsc.load_gather(asrc_v, [s16]) + dg
            a = jnp.maximum(a, 0.2 * a)
            t = maxas + dg
            e16 = jnp.exp(a - jnp.maximum(t, 0.2 * t))
            p2 = (d16 & 7) * 2
            zf = jnp.zeros((16,), jnp.float32)
            row = j * 16 + lane
            for q in range(2, 16, 2):
                plsc.store_scatter(er_ref, [row, (p2 + q) & 14], zf)
            plsc.store_scatter(er_ref, [row, p2], e16)
            return c2
        lax.fori_loop(0, CH // 16, e_body, 0)

    ceven = lax.iota(jnp.int32, 16) * 2
    codd = ceven + 1

    def mult(b, er_ref):
        def m_body(i, c2):
            ers = []
            unpacked = []
            for k in range(2):
                r = 2 * i + k
                ers.append(plsc.load_gather(
                    er_ref, [jnp.full((16,), r, jnp.int32), zero16]))
                for g in range(2):
                    unpacked.append(plsc.unpack(
                        hbrows[b, r, pl.ds(g * 32, 32)],
                        format=plsc.PackFormat.INTERLEAVED))
            for k in range(2):
                r = 2 * i + k
                fr_row = frows.at[b].at[r]
                for g in range(2):
                    ea, eb = unpacked[2 * k + g]
                    plsc.store_scatter(fr_row, [g * 32 + ceven], ea * ers[k])
                    plsc.store_scatter(fr_row, [g * 32 + codd], eb * ers[k])
            return c2
        lax.fori_loop(0, CH // 2, m_body, 0)

    def issue_gather(b):
        pltpu.async_copy(hp_sp.at[srcr.at[b]], hbrows.at[b], g_sem)

    def wait_gather(b):
        pltpu.make_async_copy(hp_sp.at[srcr.at[b]], hbrows.at[b],
                              g_sem).wait()

    def issue_scatter(b):
        pltpu.async_copy(frows.at[b], out_sp.at[dstr.at[b]], s_sem, add=True)
        pltpu.async_copy(erows.at[b], den_sp.at[dst2r.at[b]], s_sem,
                         add=True)

    def wait_scatter(b):
        pltpu.make_async_copy(frows.at[b], out_sp.at[dstr.at[b]],
                              s_sem).wait()
        pltpu.make_async_copy(erows.at[b], den_sp.at[dst2r.at[b]],
                              s_sem).wait()

    decode(0, 0)
    issue_gather(0)

    def chunk_body(ck, carry):
        b = ck & 1

        @pl.when(ck > 0)
        def _():
            wait_scatter(1 - b)

        @pl.when(ck + 1 < CPT)
        def _():
            decode(ck + 1, 1 - b)
            issue_gather(1 - b)
        e_compute(b, erows.at[b])
        wait_gather(b)
        mult(b, erows.at[b])
        issue_scatter(b)
        return carry
    lax.fori_loop(0, CPT, chunk_body, 0)
    wait_scatter((CPT - 1) & 1)

    plsc.subcore_barrier()
    pltpu.sync_copy(out_sp.at[pl.ds(rowbase, RPT)],
                    out_h.at[cid].at[pl.ds(rowbase, RPT)])
    pltpu.sync_copy(den_sp.at[pl.ds(rowbase_h, RPTH)],
                    den_h.at[cid].at[pl.ds(rowbase_h, RPTH)])


def _sc_gat(pk3d, asrc, adst, hp2):
    f = pl.kernel(
        _sc_gat_body,
        out_type=(jax.ShapeDtypeStruct((2, NP, Z_DIM), jnp.float32),
                  jax.ShapeDtypeStruct((2, NPH, 16), jnp.float32)),
        mesh=plsc.VectorSubcoreMesh(core_axis_name="c", subcore_axis_name="s"),
        compiler_params=pltpu.CompilerParams(needs_layout_passes=False,
                                             use_tc_tiling_on_sc=False),
        scratch_types=[
            pltpu.VMEM((CPT, CH), jnp.int32),
            pltpu.VMEM((2, CH), jnp.int32),
            pltpu.VMEM((2, CH), jnp.int32),
            pltpu.VMEM((2, CH), jnp.int32),
            pltpu.VMEM((NP,), jnp.float32),
            pltpu.VMEM((NP,), jnp.float32),
            pltpu.VMEM((2, CH, Z_DIM), jnp.bfloat16),
            pltpu.VMEM((2, CH, Z_DIM), jnp.float32),
            pltpu.VMEM((2, CH, 16), jnp.float32),
            pltpu.VMEM_SHARED((NP, Z_DIM), jnp.float32),
            pltpu.VMEM_SHARED((NPH, 16), jnp.float32),
            pltpu.VMEM_SHARED((NHP, Z_DIM), jnp.bfloat16),
            pltpu.SemaphoreType.DMA,
            pltpu.SemaphoreType.DMA,
        ],
    )
    return f(pk3d, asrc, adst, hp2)


def _tc2_body(out_ref, den_ref, b_ref, w_ref, as_ref, ad_ref,
              h_ref, s_ref, d_ref):
    agg = out_ref[0] + out_ref[1]
    den = den_ref[...]
    x = _gelu(agg / (den + 1e-16) + b_ref[...])
    h = jnp.dot(x, w_ref[...], preferred_element_type=jnp.float32)
    h_ref[...] = h.astype(jnp.bfloat16)
    s_ref[...] = jnp.sum(h * as_ref[...], axis=1)
    d_ref[...] = jnp.sum(h * ad_ref[...], axis=1)


def _tc2(out1, den1, b, W, a_s, a_d):
    grid = ((N + _BLK - 1) // _BLK,)
    return pl.pallas_call(
        _tc2_body,
        grid=grid,
        in_specs=[
            pl.BlockSpec((2, _BLK, Z_DIM), lambda i: (0, i, 0)),
            pl.BlockSpec((_BLK, 1), lambda i: (i, 0)),
            pl.BlockSpec((1, Z_DIM), lambda i: (0, 0)),
            pl.BlockSpec((Z_DIM, Z_DIM), lambda i: (0, 0)),
            pl.BlockSpec((1, Z_DIM), lambda i: (0, 0)),
            pl.BlockSpec((1, Z_DIM), lambda i: (0, 0)),
        ],
        out_specs=[
            pl.BlockSpec((_BLK, Z_DIM), lambda i: (i, 0)),
            pl.BlockSpec((_BLK,), lambda i: (i,)),
            pl.BlockSpec((_BLK,), lambda i: (i,)),
        ],
        out_shape=[
            jax.ShapeDtypeStruct((N, Z_DIM), jnp.bfloat16),
            jax.ShapeDtypeStruct((N,), jnp.float32),
            jax.ShapeDtypeStruct((N,), jnp.float32),
        ],
    )(out1, den1, b.reshape(1, Z_DIM), W,
      a_s.reshape(1, Z_DIM), a_d.reshape(1, Z_DIM))


def _tc3_body(out_ref, den_ref, b_ref, gcw_ref, gcb_ref, genw_ref, genb_ref,
              decw_ref, decb_ref, x_ref, acc_ref):
    i = pl.program_id(0)
    agg = out_ref[0] + out_ref[1]
    den = den_ref[...]
    z = _gelu(agg / (den + 1e-16) + b_ref[...])
    z = jnp.dot(z, gcw_ref[...], preferred_element_type=jnp.float32) + gcb_ref[...]
    z = jnp.dot(z, genw_ref[...], preferred_element_type=jnp.float32) + genb_ref[...]
    xh = jnp.dot(z, decw_ref[...], preferred_element_type=jnp.float32) + decb_ref[...]
    d = xh - x_ref[...]
    rows = i * _BLK + lax.broadcasted_iota(jnp.int32, (_BLK, IN_DIM), 0)
    d = jnp.where(rows < N, d, 0.0)
    s = jnp.sum(d * d).reshape(1, 1)

    @pl.when(i == 0)
    def _():
        acc_ref[...] = jnp.zeros((1, 1), jnp.float32)
    acc_ref[...] += s


def _tc3(out2, den2, b, gcW, gcb, genW, genb, decW, decb, X):
    grid = ((N + _BLK - 1) // _BLK,)
    return pl.pallas_call(
        _tc3_body,
        grid=grid,
        in_specs=[
            pl.BlockSpec((2, _BLK, Z_DIM), lambda i: (0, i, 0)),
            pl.BlockSpec((_BLK, 1), lambda i: (i, 0)),
            pl.BlockSpec((1, Z_DIM), lambda i: (0, 0)),
            pl.BlockSpec((Z_DIM, Z_DIM), lambda i: (0, 0)),
            pl.BlockSpec((1, Z_DIM), lambda i: (0, 0)),
            pl.BlockSpec((Z_DIM, Z_DIM), lambda i: (0, 0)),
            pl.BlockSpec((1, Z_DIM), lambda i: (0, 0)),
            pl.BlockSpec((Z_DIM, IN_DIM), lambda i: (0, 0)),
            pl.BlockSpec((1, IN_DIM), lambda i: (0, 0)),
            pl.BlockSpec((_BLK, IN_DIM), lambda i: (i, 0)),
        ],
        out_specs=pl.BlockSpec((1, 1), lambda i: (0, 0)),
        out_shape=jax.ShapeDtypeStruct((1, 1), jnp.float32),
    )(out2, den2, b.reshape(1, Z_DIM), gcW, gcb.reshape(1, Z_DIM),
      genW, genb.reshape(1, Z_DIM), decW, decb.reshape(1, IN_DIM), X)


def _shift_and_pad(a_s, a_d):
    pad = NP - N
    asrc = jnp.pad(a_s, (0, pad)).at[NP - 1].set(jnp.max(a_s))
    return asrc, jnp.pad(a_d, (0, pad))


def _unpack_den(den):
    return (den[0] + den[1]).reshape(NPH * 8, 2)[:, 0:1]


def kernel(X, edge_index, edge_weight, fn_W1, fn_b1, fn_W2, fn_b2,
           gat1_W, gat1_as, gat1_ad, gat1_b,
           gat2_W, gat2_as, gat2_ad, gat2_b,
           gc_W, gc_b, gen_W, gen_b, dec_W, dec_b):
    loop = jnp.arange(N, dtype=jnp.int32)
    padi = jnp.full((EP - EA,), N, jnp.int32)
    src_all = jnp.concatenate([edge_index[0], loop, padi])
    dst_all = jnp.concatenate([edge_index[1], loop, padi])
    pk3d = (src_all | (dst_all << 14)).reshape(NWORK, CPT, CH)
    pad = NP - N

    h1, as1, ad1 = _tc1(X, fn_W1, fn_b1, fn_W2, fn_b2, gat1_W, gat1_as, gat1_ad)
    asrc1, adst1 = _shift_and_pad(as1, ad1)
    hp1 = jnp.pad(h1, ((0, NHP - N), (0, 0)))
    out1, den1 = _sc_gat(pk3d, asrc1, adst1, hp1)

    h2, as2, ad2 = _tc2(out1, _unpack_den(den1), gat1_b, gat2_W, gat2_as, gat2_ad)
    asrc2, adst2 = _shift_and_pad(as2, ad2)
    hp2 = jnp.pad(h2, ((0, NHP - N), (0, 0)))
    out2, den2 = _sc_gat(pk3d, asrc2, adst2, hp2)

    acc = _tc3(out2, _unpack_den(den2), gat2_b, gc_W, gc_b, gen_W, gen_b,
               dec_W, dec_b, X)
    return acc[0, 0] / float(N * IN_DIM)


# speedup vs baseline: 1.8757x; 1.0382x over previous
import jax
import jax.numpy as jnp
from jax import lax
from jax.experimental import pallas as pl
from jax.experimental.pallas import tpu as pltpu
from jax.experimental.pallas import tpu_sc as plsc

N = 10000
IN_DIM = 128
H_DIM = 128
Z_DIM = 64
HW = Z_DIM // 2

NP = 10016
E = 320000
EA = E + N
CH = 128
NWORK = 32
CPT = 81
EP = NWORK * CPT * CH
RPT = NP // 16
NPH = 1264
RPTH = NPH // 16
NHP = 10016
RHPT = NHP // 16

_BLK = 1024


def _gelu(x):
    return 0.5 * x * (1.0 + jax.lax.erf(x * 0.7071067811865476))


def _tc1_body(x_ref, w1_ref, b1_ref, w2_ref, b2_ref, gw_ref, as_ref, ad_ref,
              h_ref, s_ref, d_ref):
    z = _gelu(jnp.dot(x_ref[...], w1_ref[...],
                      preferred_element_type=jnp.float32) + b1_ref[...])
    z = _gelu(jnp.dot(z, w2_ref[...],
                      preferred_element_type=jnp.float32) + b2_ref[...])
    h = jnp.dot(z, gw_ref[...], preferred_element_type=jnp.float32)
    h_ref[...] = h.astype(jnp.bfloat16)
    s_ref[...] = jnp.sum(h * as_ref[...], axis=1)
    d_ref[...] = jnp.sum(h * ad_ref[...], axis=1)


def _tc1(X, W1, b1, W2, b2, gW, a_s, a_d):
    grid = ((N + _BLK - 1) // _BLK,)
    return pl.pallas_call(
        _tc1_body,
        grid=grid,
        in_specs=[
            pl.BlockSpec((_BLK, IN_DIM), lambda i: (i, 0)),
            pl.BlockSpec((IN_DIM, H_DIM), lambda i: (0, 0)),
            pl.BlockSpec((H_DIM,), lambda i: (0,)),
            pl.BlockSpec((H_DIM, H_DIM), lambda i: (0, 0)),
            pl.BlockSpec((H_DIM,), lambda i: (0,)),
            pl.BlockSpec((H_DIM, Z_DIM), lambda i: (0, 0)),
            pl.BlockSpec((1, Z_DIM), lambda i: (0, 0)),
            pl.BlockSpec((1, Z_DIM), lambda i: (0, 0)),
        ],
        out_specs=[
            pl.BlockSpec((_BLK, Z_DIM), lambda i: (i, 0)),
            pl.BlockSpec((_BLK,), lambda i: (i,)),
            pl.BlockSpec((_BLK,), lambda i: (i,)),
        ],
        out_shape=[
            jax.ShapeDtypeStruct((N, Z_DIM), jnp.bfloat16),
            jax.ShapeDtypeStruct((N,), jnp.float32),
            jax.ShapeDtypeStruct((N,), jnp.float32),
        ],
    )(X, W1, b1, W2, b2, gW, a_s.reshape(1, Z_DIM), a_d.reshape(1, Z_DIM))


def _sc_gat_body(pk_h, asrc_h, adst_h, hp_h,
                 out_h, den_h,
                 pk_v, srcr, dstr, dst2r, asrc_v, adst_v,
                 hbrows, frows, erows,
                 out_sp, den_sp, hp_sp, g_sem, s_sem):
    cid = lax.axis_index("c")
    sid = lax.axis_index("s")
    rowbase = sid * RPT

    pltpu.sync_copy(asrc_h, asrc_v)
    pltpu.sync_copy(adst_h, adst_v)
    pltpu.sync_copy(pk_h.at[cid * 16 + sid], pk_v)
    maxas = plsc.load_gather(asrc_v, [jnp.full((16,), NP - 1, jnp.int32)])
    pltpu.sync_copy(hp_h.at[pl.ds(sid * RHPT, RHPT)],
                    hp_sp.at[pl.ds(sid * RHPT, RHPT)])

    def zrow(r, carry):
        for b in range(2):
            for g in range(Z_DIM // 16):
                frows[b, r, pl.ds(g * 16, 16)] = jnp.zeros((16,), jnp.float32)
            erows[b, r, :] = jnp.zeros((16,), jnp.float32)
        return carry
    lax.fori_loop(0, CH, zrow, 0)

    def zcopy(k, carry):
        pltpu.sync_copy(frows.at[0].at[pl.ds(0, 128)],
                        out_sp.at[pl.ds(rowbase + k * 128, 128)])
        return carry
    lax.fori_loop(0, 4, zcopy, 0)
    pltpu.sync_copy(frows.at[0].at[pl.ds(0, 114)],
                    out_sp.at[pl.ds(rowbase + 512, 114)])
    rowbase_h = sid * RPTH
    pltpu.sync_copy(erows.at[0].at[pl.ds(0, RPTH)],
                    den_sp.at[pl.ds(rowbase_h, RPTH)])

    plsc.subcore_barrier()

    zero16 = jnp.zeros((16,), jnp.int32)
    lane = lax.iota(jnp.int32, 16)

    def decode(ck, b):
        def d_body(j, c2):
            sl = pl.ds(j * 16, 16)
            p = pk_v[ck, sl]
            d16 = lax.shift_right_logical(p, 14)
            srcr[b, sl] = p & 16383
            dstr[b, sl] = d16
            dst2r[b, sl] = lax.shift_right_logical(d16, 3)
            return c2
        lax.fori_loop(0, CH // 16, d_body, 0)

    def e_compute(b, er_ref):
        def e_body(j, c2):
            s16 = srcr[b, pl.ds(j * 16, 16)]
            d16 = dstr[b, pl.ds(j * 16, 16)]
            dg = plsc.load_gather(adst_v, [d16])
            a = plsc.load_gather(asrc_v, [s16]) + dg
            a = jnp.maximum(a, 0.2 * a)
            t = maxas + dg
            e16 = jnp.exp(a - jnp.maximum(t, 0.2 * t))
            p2 = (d16 & 7) * 2
            zf = jnp.zeros((16,), jnp.float32)
            row = j * 16 + lane
            for q in range(2, 16, 2):
                plsc.store_scatter(er_ref, [row, (p2 + q) & 14], zf)
            plsc.store_scatter(er_ref, [row, p2], e16)
            return c2
        lax.fori_loop(0, CH // 16, e_body, 0)

    ceven = lax.iota(jnp.int32, 16) * 2
    codd = ceven + 1

    def mult(b, er_ref):
        def m_body(i, c2):
            ers = []
            unpacked = []
            for k in range(4):
                r = 4 * i + k
                ers.append(plsc.load_gather(
                    er_ref, [jnp.full((16,), r, jnp.int32), zero16]))
                for g in range(2):
                    unpacked.append(plsc.unpack(
                        hbrows[b, r, pl.ds(g * 32, 32)],
                        format=plsc.PackFormat.INTERLEAVED))
            for k in range(4):
                r = 4 * i + k
                fr_row = frows.at[b].at[r]
                for g in range(2):
                    ea, eb = unpacked[2 * k + g]
                    plsc.store_scatter(fr_row, [g * 32 + ceven], ea * ers[k])
                    plsc.store_scatter(fr_row, [g * 32 + codd], eb * ers[k])
            return c2
        lax.fori_loop(0, CH // 4, m_body, 0)

    def issue_gather(b):
        pltpu.async_copy(hp_sp.at[srcr.at[b]], hbrows.at[b], g_sem)

    def wait_gather(b):
        pltpu.make_async_copy(hp_sp.at[srcr.at[b]], hbrows.at[b],
                              g_sem).wait()

    def issue_scatter(b):
        pltpu.async_copy(frows.at[b], out_sp.at[dstr.at[b]], s_sem, add=True)
        pltpu.async_copy(erows.at[b], den_sp.at[dst2r.at[b]], s_sem,
                         add=True)

    def wait_scatter(b):
        pltpu.make_async_copy(frows.at[b], out_sp.at[dstr.at[b]],
                              s_sem).wait()
        pltpu.make_async_copy(erows.at[b], den_sp.at[dst2r.at[b]],
                              s_sem).wait()

    decode(0, 0)
    issue_gather(0)

    def chunk_body(ck, carry):
        b = ck & 1

        @pl.when(ck > 0)
        def _():
            wait_scatter(1 - b)

        @pl.when(ck + 1 < CPT)
        def _():
            decode(ck + 1, 1 - b)
            issue_gather(1 - b)
        e_compute(b, erows.at[b])
        wait_gather(b)
        mult(b, erows.at[b])
        issue_scatter(b)
        return carry
    lax.fori_loop(0, CPT, chunk_body, 0)
    wait_scatter((CPT - 1) & 1)

    plsc.subcore_barrier()
    pltpu.sync_copy(out_sp.at[pl.ds(rowbase, RPT)],
                    out_h.at[cid].at[pl.ds(rowbase, RPT)])
    pltpu.sync_copy(den_sp.at[pl.ds(rowbase_h, RPTH)],
                    den_h.at[cid].at[pl.ds(rowbase_h, RPTH)])


def _sc_gat(pk3d, asrc, adst, hp2):
    f = pl.kernel(
        _sc_gat_body,
        out_type=(jax.ShapeDtypeStruct((2, NP, Z_DIM), jnp.float32),
                  jax.ShapeDtypeStruct((2, NPH, 16), jnp.float32)),
        mesh=plsc.VectorSubcoreMesh(core_axis_name="c", subcore_axis_name="s"),
        compiler_params=pltpu.CompilerParams(needs_layout_passes=False,
                                             use_tc_tiling_on_sc=False),
        scratch_types=[
            pltpu.VMEM((CPT, CH), jnp.int32),
            pltpu.VMEM((2, CH), jnp.int32),
            pltpu.VMEM((2, CH), jnp.int32),
            pltpu.VMEM((2, CH), jnp.int32),
            pltpu.VMEM((NP,), jnp.float32),
            pltpu.VMEM((NP,), jnp.float32),
            pltpu.VMEM((2, CH, Z_DIM), jnp.bfloat16),
            pltpu.VMEM((2, CH, Z_DIM), jnp.float32),
            pltpu.VMEM((2, CH, 16), jnp.float32),
            pltpu.VMEM_SHARED((NP, Z_DIM), jnp.float32),
            pltpu.VMEM_SHARED((NPH, 16), jnp.float32),
            pltpu.VMEM_SHARED((NHP, Z_DIM), jnp.bfloat16),
            pltpu.SemaphoreType.DMA,
            pltpu.SemaphoreType.DMA,
        ],
    )
    return f(pk3d, asrc, adst, hp2)


def _tc2_body(out_ref, den_ref, b_ref, w_ref, as_ref, ad_ref,
              h_ref, s_ref, d_ref):
    agg = out_ref[0] + out_ref[1]
    den = den_ref[...]
    x = _gelu(agg / (den + 1e-16) + b_ref[...])
    h = jnp.dot(x, w_ref[...], preferred_element_type=jnp.float32)
    h_ref[...] = h.astype(jnp.bfloat16)
    s_ref[...] = jnp.sum(h * as_ref[...], axis=1)
    d_ref[...] = jnp.sum(h * ad_ref[...], axis=1)


def _tc2(out1, den1, b, W, a_s, a_d):
    grid = ((N + _BLK - 1) // _BLK,)
    return pl.pallas_call(
        _tc2_body,
        grid=grid,
        in_specs=[
            pl.BlockSpec((2, _BLK, Z_DIM), lambda i: (0, i, 0)),
            pl.BlockSpec((_BLK, 1), lambda i: (i, 0)),
            pl.BlockSpec((1, Z_DIM), lambda i: (0, 0)),
            pl.BlockSpec((Z_DIM, Z_DIM), lambda i: (0, 0)),
            pl.BlockSpec((1, Z_DIM), lambda i: (0, 0)),
            pl.BlockSpec((1, Z_DIM), lambda i: (0, 0)),
        ],
        out_specs=[
            pl.BlockSpec((_BLK, Z_DIM), lambda i: (i, 0)),
            pl.BlockSpec((_BLK,), lambda i: (i,)),
            pl.BlockSpec((_BLK,), lambda i: (i,)),
        ],
        out_shape=[
            jax.ShapeDtypeStruct((N, Z_DIM), jnp.bfloat16),
            jax.ShapeDtypeStruct((N,), jnp.float32),
            jax.ShapeDtypeStruct((N,), jnp.float32),
        ],
    )(out1, den1, b.reshape(1, Z_DIM), W,
      a_s.reshape(1, Z_DIM), a_d.reshape(1, Z_DIM))


def _tc3_body(out_ref, den_ref, b_ref, gcw_ref, gcb_ref, genw_ref, genb_ref,
              decw_ref, decb_ref, x_ref, acc_ref):
    i = pl.program_id(0)
    agg = out_ref[0] + out_ref[1]
    den = den_ref[...]
    z = _gelu(agg / (den + 1e-16) + b_ref[...])
    z = jnp.dot(z, gcw_ref[...], preferred_element_type=jnp.float32) + gcb_ref[...]
    z = jnp.dot(z, genw_ref[...], preferred_element_type=jnp.float32) + genb_ref[...]
    xh = jnp.dot(z, decw_ref[...], preferred_element_type=jnp.float32) + decb_ref[...]
    d = xh - x_ref[...]
    rows = i * _BLK + lax.broadcasted_iota(jnp.int32, (_BLK, IN_DIM), 0)
    d = jnp.where(rows < N, d, 0.0)
    s = jnp.sum(d * d).reshape(1, 1)

    @pl.when(i == 0)
    def _():
        acc_ref[...] = jnp.zeros((1, 1), jnp.float32)
    acc_ref[...] += s


def _tc3(out2, den2, b, gcW, gcb, genW, genb, decW, decb, X):
    grid = ((N + _BLK - 1) // _BLK,)
    return pl.pallas_call(
        _tc3_body,
        grid=grid,
        in_specs=[
            pl.BlockSpec((2, _BLK, Z_DIM), lambda i: (0, i, 0)),
            pl.BlockSpec((_BLK, 1), lambda i: (i, 0)),
            pl.BlockSpec((1, Z_DIM), lambda i: (0, 0)),
            pl.BlockSpec((Z_DIM, Z_DIM), lambda i: (0, 0)),
            pl.BlockSpec((1, Z_DIM), lambda i: (0, 0)),
            pl.BlockSpec((Z_DIM, Z_DIM), lambda i: (0, 0)),
            pl.BlockSpec((1, Z_DIM), lambda i: (0, 0)),
            pl.BlockSpec((Z_DIM, IN_DIM), lambda i: (0, 0)),
            pl.BlockSpec((1, IN_DIM), lambda i: (0, 0)),
            pl.BlockSpec((_BLK, IN_DIM), lambda i: (i, 0)),
        ],
        out_specs=pl.BlockSpec((1, 1), lambda i: (0, 0)),
        out_shape=jax.ShapeDtypeStruct((1, 1), jnp.float32),
    )(out2, den2, b.reshape(1, Z_DIM), gcW, gcb.reshape(1, Z_DIM),
      genW, genb.reshape(1, Z_DIM), decW, decb.reshape(1, IN_DIM), X)


def _shift_and_pad(a_s, a_d):
    pad = NP - N
    asrc = jnp.pad(a_s, (0, pad)).at[NP - 1].set(jnp.max(a_s))
    return asrc, jnp.pad(a_d, (0, pad))


def _unpack_den(den):
    return (den[0] + den[1]).reshape(NPH * 8, 2)[:, 0:1]


def kernel(X, edge_index, edge_weight, fn_W1, fn_b1, fn_W2, fn_b2,
           gat1_W, gat1_as, gat1_ad, gat1_b,
           gat2_W, gat2_as, gat2_ad, gat2_b,
           gc_W, gc_b, gen_W, gen_b, dec_W, dec_b):
    loop = jnp.arange(N, dtype=jnp.int32)
    padi = jnp.full((EP - EA,), N, jnp.int32)
    src_all = jnp.concatenate([edge_index[0], loop, padi])
    dst_all = jnp.concatenate([edge_index[1], loop, padi])
    pk3d = (src_all | (dst_all << 14)).reshape(NWORK, CPT, CH)
    pad = NP - N

    h1, as1, ad1 = _tc1(X, fn_W1, fn_b1, fn_W2, fn_b2, gat1_W, gat1_as, gat1_ad)
    asrc1, adst1 = _shift_and_pad(as1, ad1)
    hp1 = jnp.pad(h1, ((0, NHP - N), (0, 0)))
    out1, den1 = _sc_gat(pk3d, asrc1, adst1, hp1)

    h2, as2, ad2 = _tc2(out1, _unpack_den(den1), gat1_b, gat2_W, gat2_as, gat2_ad)
    asrc2, adst2 = _shift_and_pad(as2, ad2)
    hp2 = jnp.pad(h2, ((0, NHP - N), (0, 0)))
    out2, den2 = _sc_gat(pk3d, asrc2, adst2, hp2)

    acc = _tc3(out2, _unpack_den(den2), gat2_b, gc_W, gc_b, gen_W, gen_b,
               dec_W, dec_b, X)
    return acc[0, 0] / float(N * IN_DIM)


# speedup vs baseline: 1.8831x; 1.0039x over previous
import jax
import jax.numpy as jnp
from jax import lax
from jax.experimental import pallas as pl
from jax.experimental.pallas import tpu as pltpu
from jax.experimental.pallas import tpu_sc as plsc

N = 10000
IN_DIM = 128
H_DIM = 128
Z_DIM = 64
HW = Z_DIM // 2

NP = 10016
E = 320000
EA = E + N
CH = 128
NWORK = 32
CPT = 81
EP = NWORK * CPT * CH
RPT = NP // 16
NPH = 1264
RPTH = NPH // 16
NHP = 10016
RHPT = NHP // 16

_BLK = 1024


def _gelu(x):
    return 0.5 * x * (1.0 + jax.lax.erf(x * 0.7071067811865476))


def _tc1_body(x_ref, w1_ref, b1_ref, w2_ref, b2_ref, gw_ref, as_ref, ad_ref,
              h_ref, s_ref, d_ref):
    z = _gelu(jnp.dot(x_ref[...], w1_ref[...],
                      preferred_element_type=jnp.float32) + b1_ref[...])
    z = _gelu(jnp.dot(z, w2_ref[...],
                      preferred_element_type=jnp.float32) + b2_ref[...])
    h = jnp.dot(z, gw_ref[...], preferred_element_type=jnp.float32)
    h_ref[...] = h.astype(jnp.bfloat16)
    s_ref[...] = jnp.sum(h * as_ref[...], axis=1)
    d_ref[...] = jnp.sum(h * ad_ref[...], axis=1)


def _tc1(X, W1, b1, W2, b2, gW, a_s, a_d):
    grid = ((N + _BLK - 1) // _BLK,)
    return pl.pallas_call(
        _tc1_body,
        grid=grid,
        in_specs=[
            pl.BlockSpec((_BLK, IN_DIM), lambda i: (i, 0)),
            pl.BlockSpec((IN_DIM, H_DIM), lambda i: (0, 0)),
            pl.BlockSpec((H_DIM,), lambda i: (0,)),
            pl.BlockSpec((H_DIM, H_DIM), lambda i: (0, 0)),
            pl.BlockSpec((H_DIM,), lambda i: (0,)),
            pl.BlockSpec((H_DIM, Z_DIM), lambda i: (0, 0)),
            pl.BlockSpec((1, Z_DIM), lambda i: (0, 0)),
            pl.BlockSpec((1, Z_DIM), lambda i: (0, 0)),
        ],
        out_specs=[
            pl.BlockSpec((_BLK, Z_DIM), lambda i: (i, 0)),
            pl.BlockSpec((_BLK,), lambda i: (i,)),
            pl.BlockSpec((_BLK,), lambda i: (i,)),
        ],
        out_shape=[
            jax.ShapeDtypeStruct((NHP, Z_DIM), jnp.bfloat16),
            jax.ShapeDtypeStruct((N,), jnp.float32),
            jax.ShapeDtypeStruct((N,), jnp.float32),
        ],
    )(X, W1, b1, W2, b2, gW, a_s.reshape(1, Z_DIM), a_d.reshape(1, Z_DIM))


def _sc_gat_body(pk_h, asrc_h, adst_h, hp_h,
                 out_h, den_h,
                 pk_v, srcr, dstr, dst2r, asrc_v, adst_v,
                 hbrows, frows, erows,
                 out_sp, den_sp, hp_sp, g_sem, s_sem):
    cid = lax.axis_index("c")
    sid = lax.axis_index("s")
    rowbase = sid * RPT

    pltpu.sync_copy(asrc_h, asrc_v)
    pltpu.sync_copy(adst_h, adst_v)
    pltpu.sync_copy(pk_h.at[cid * 16 + sid], pk_v)
    maxas = plsc.load_gather(asrc_v, [jnp.full((16,), NP - 1, jnp.int32)])
    pltpu.sync_copy(hp_h.at[pl.ds(sid * RHPT, RHPT)],
                    hp_sp.at[pl.ds(sid * RHPT, RHPT)])

    def zrow(r, carry):
        for b in range(2):
            for g in range(Z_DIM // 16):
                frows[b, r, pl.ds(g * 16, 16)] = jnp.zeros((16,), jnp.float32)
            erows[b, r, :] = jnp.zeros((16,), jnp.float32)
        return carry
    lax.fori_loop(0, CH, zrow, 0)

    def zcopy(k, carry):
        pltpu.sync_copy(frows.at[0].at[pl.ds(0, 128)],
                        out_sp.at[pl.ds(rowbase + k * 128, 128)])
        return carry
    lax.fori_loop(0, 4, zcopy, 0)
    pltpu.sync_copy(frows.at[0].at[pl.ds(0, 114)],
                    out_sp.at[pl.ds(rowbase + 512, 114)])
    rowbase_h = sid * RPTH
    pltpu.sync_copy(erows.at[0].at[pl.ds(0, RPTH)],
                    den_sp.at[pl.ds(rowbase_h, RPTH)])

    plsc.subcore_barrier()

    zero16 = jnp.zeros((16,), jnp.int32)
    lane = lax.iota(jnp.int32, 16)

    def decode(ck, b):
        def d_body(j, c2):
            sl = pl.ds(j * 16, 16)
            p = pk_v[ck, sl]
            d16 = lax.shift_right_logical(p, 14)
            srcr[b, sl] = p & 16383
            dstr[b, sl] = d16
            dst2r[b, sl] = lax.shift_right_logical(d16, 3)
            return c2
        lax.fori_loop(0, CH // 16, d_body, 0)

    def e_compute(b, er_ref):
        def e_body(i, c2):
            for k in range(2):
                j = 2 * i + k
                s16 = srcr[b, pl.ds(j * 16, 16)]
                d16 = dstr[b, pl.ds(j * 16, 16)]
                dg = plsc.load_gather(adst_v, [d16])
                a = plsc.load_gather(asrc_v, [s16]) + dg
                a = jnp.maximum(a, 0.2 * a)
                t = maxas + dg
                e16 = jnp.exp(a - jnp.maximum(t, 0.2 * t))
                p2 = (d16 & 7) * 2
                zf = jnp.zeros((16,), jnp.float32)
                row = j * 16 + lane
                for q in range(2, 16, 2):
                    plsc.store_scatter(er_ref, [row, (p2 + q) & 14], zf)
                plsc.store_scatter(er_ref, [row, p2], e16)
            return c2
        lax.fori_loop(0, CH // 32, e_body, 0)

    ceven = lax.iota(jnp.int32, 16) * 2
    codd = ceven + 1

    def mult(b, er_ref):
        def m_body(i, c2):
            ers = []
            unpacked = []
            for k in range(4):
                r = 4 * i + k
                ers.append(plsc.load_gather(
                    er_ref, [jnp.full((16,), r, jnp.int32), zero16]))
                for g in range(2):
                    unpacked.append(plsc.unpack(
                        hbrows[b, r, pl.ds(g * 32, 32)],
                        format=plsc.PackFormat.INTERLEAVED))
            for k in range(4):
                r = 4 * i + k
                fr_row = frows.at[b].at[r]
                for g in range(2):
                    ea, eb = unpacked[2 * k + g]
                    plsc.store_scatter(fr_row, [g * 32 + ceven], ea * ers[k])
                    plsc.store_scatter(fr_row, [g * 32 + codd], eb * ers[k])
            return c2
        lax.fori_loop(0, CH // 4, m_body, 0)

    def issue_gather(b):
        pltpu.async_copy(hp_sp.at[srcr.at[b]], hbrows.at[b], g_sem)

    def wait_gather(b):
        pltpu.make_async_copy(hp_sp.at[srcr.at[b]], hbrows.at[b],
                              g_sem).wait()

    def issue_scatter(b):
        pltpu.async_copy(frows.at[b], out_sp.at[dstr.at[b]], s_sem, add=True)
        pltpu.async_copy(erows.at[b], den_sp.at[dst2r.at[b]], s_sem,
                         add=True)

    def wait_scatter(b):
        pltpu.make_async_copy(frows.at[b], out_sp.at[dstr.at[b]],
                              s_sem).wait()
        pltpu.make_async_copy(erows.at[b], den_sp.at[dst2r.at[b]],
                              s_sem).wait()

    decode(0, 0)
    issue_gather(0)

    def chunk_body(ck, carry):
        b = ck & 1

        @pl.when(ck > 0)
        def _():
            wait_scatter(1 - b)

        @pl.when(ck + 1 < CPT)
        def _():
            decode(ck + 1, 1 - b)
            issue_gather(1 - b)
        e_compute(b, erows.at[b])
        wait_gather(b)
        mult(b, erows.at[b])
        issue_scatter(b)
        return carry
    lax.fori_loop(0, CPT, chunk_body, 0)
    wait_scatter((CPT - 1) & 1)

    plsc.subcore_barrier()
    pltpu.sync_copy(out_sp.at[pl.ds(rowbase, RPT)],
                    out_h.at[cid].at[pl.ds(rowbase, RPT)])
    pltpu.sync_copy(den_sp.at[pl.ds(rowbase_h, RPTH)],
                    den_h.at[cid].at[pl.ds(rowbase_h, RPTH)])


def _sc_gat(pk3d, asrc, adst, hp2):
    f = pl.kernel(
        _sc_gat_body,
        out_type=(jax.ShapeDtypeStruct((2, NP, Z_DIM), jnp.float32),
                  jax.ShapeDtypeStruct((2, NPH, 16), jnp.float32)),
        mesh=plsc.VectorSubcoreMesh(core_axis_name="c", subcore_axis_name="s"),
        compiler_params=pltpu.CompilerParams(needs_layout_passes=False,
                                             use_tc_tiling_on_sc=False),
        scratch_types=[
            pltpu.VMEM((CPT, CH), jnp.int32),
            pltpu.VMEM((2, CH), jnp.int32),
            pltpu.VMEM((2, CH), jnp.int32),
            pltpu.VMEM((2, CH), jnp.int32),
            pltpu.VMEM((NP,), jnp.float32),
            pltpu.VMEM((NP,), jnp.float32),
            pltpu.VMEM((2, CH, Z_DIM), jnp.bfloat16),
            pltpu.VMEM((2, CH, Z_DIM), jnp.float32),
            pltpu.VMEM((2, CH, 16), jnp.float32),
            pltpu.VMEM_SHARED((NP, Z_DIM), jnp.float32),
            pltpu.VMEM_SHARED((NPH, 16), jnp.float32),
            pltpu.VMEM_SHARED((NHP, Z_DIM), jnp.bfloat16),
            pltpu.SemaphoreType.DMA,
            pltpu.SemaphoreType.DMA,
        ],
    )
    return f(pk3d, asrc, adst, hp2)


def _tc2_body(out_ref, den_ref, b_ref, w_ref, as_ref, ad_ref,
              h_ref, s_ref, d_ref):
    agg = out_ref[0] + out_ref[1]
    den = den_ref[...]
    x = _gelu(agg / (den + 1e-16) + b_ref[...])
    h = jnp.dot(x, w_ref[...], preferred_element_type=jnp.float32)
    h_ref[...] = h.astype(jnp.bfloat16)
    s_ref[...] = jnp.sum(h * as_ref[...], axis=1)
    d_ref[...] = jnp.sum(h * ad_ref[...], axis=1)


def _tc2(out1, den1, b, W, a_s, a_d):
    grid = ((N + _BLK - 1) // _BLK,)
    return pl.pallas_call(
        _tc2_body,
        grid=grid,
        in_specs=[
            pl.BlockSpec((2, _BLK, Z_DIM), lambda i: (0, i, 0)),
            pl.BlockSpec((_BLK, 1), lambda i: (i, 0)),
            pl.BlockSpec((1, Z_DIM), lambda i: (0, 0)),
            pl.BlockSpec((Z_DIM, Z_DIM), lambda i: (0, 0)),
            pl.BlockSpec((1, Z_DIM), lambda i: (0, 0)),
            pl.BlockSpec((1, Z_DIM), lambda i: (0, 0)),
        ],
        out_specs=[
            pl.BlockSpec((_BLK, Z_DIM), lambda i: (i, 0)),
            pl.BlockSpec((_BLK,), lambda i: (i,)),
            pl.BlockSpec((_BLK,), lambda i: (i,)),
        ],
        out_shape=[
            jax.ShapeDtypeStruct((NHP, Z_DIM), jnp.bfloat16),
            jax.ShapeDtypeStruct((N,), jnp.float32),
            jax.ShapeDtypeStruct((N,), jnp.float32),
        ],
    )(out1, den1, b.reshape(1, Z_DIM), W,
      a_s.reshape(1, Z_DIM), a_d.reshape(1, Z_DIM))


def _tc3_body(out_ref, den_ref, b_ref, gcw_ref, gcb_ref, genw_ref, genb_ref,
              decw_ref, decb_ref, x_ref, acc_ref):
    i = pl.program_id(0)
    agg = out_ref[0] + out_ref[1]
    den = den_ref[...]
    z = _gelu(agg / (den + 1e-16) + b_ref[...])
    z = jnp.dot(z, gcw_ref[...], preferred_element_type=jnp.float32) + gcb_ref[...]
    z = jnp.dot(z, genw_ref[...], preferred_element_type=jnp.float32) + genb_ref[...]
    xh = jnp.dot(z, decw_ref[...], preferred_element_type=jnp.float32) + decb_ref[...]
    d = xh - x_ref[...]
    rows = i * _BLK + lax.broadcasted_iota(jnp.int32, (_BLK, IN_DIM), 0)
    d = jnp.where(rows < N, d, 0.0)
    s = jnp.sum(d * d).reshape(1, 1)

    @pl.when(i == 0)
    def _():
        acc_ref[...] = jnp.zeros((1, 1), jnp.float32)
    acc_ref[...] += s


def _tc3(out2, den2, b, gcW, gcb, genW, genb, decW, decb, X):
    grid = ((N + _BLK - 1) // _BLK,)
    return pl.pallas_call(
        _tc3_body,
        grid=grid,
        in_specs=[
            pl.BlockSpec((2, _BLK, Z_DIM), lambda i: (0, i, 0)),
            pl.BlockSpec((_BLK, 1), lambda i: (i, 0)),
            pl.BlockSpec((1, Z_DIM), lambda i: (0, 0)),
            pl.BlockSpec((Z_DIM, Z_DIM), lambda i: (0, 0)),
            pl.BlockSpec((1, Z_DIM), lambda i: (0, 0)),
            pl.BlockSpec((Z_DIM, Z_DIM), lambda i: (0, 0)),
            pl.BlockSpec((1, Z_DIM), lambda i: (0, 0)),
            pl.BlockSpec((Z_DIM, IN_DIM), lambda i: (0, 0)),
            pl.BlockSpec((1, IN_DIM), lambda i: (0, 0)),
            pl.BlockSpec((_BLK, IN_DIM), lambda i: (i, 0)),
        ],
        out_specs=pl.BlockSpec((1, 1), lambda i: (0, 0)),
        out_shape=jax.ShapeDtypeStruct((1, 1), jnp.float32),
    )(out2, den2, b.reshape(1, Z_DIM), gcW, gcb.reshape(1, Z_DIM),
      genW, genb.reshape(1, Z_DIM), decW, decb.reshape(1, IN_DIM), X)


def _shift_and_pad(a_s, a_d):
    pad = NP - N
    asrc = jnp.pad(a_s, (0, pad)).at[NP - 1].set(jnp.max(a_s))
    return asrc, jnp.pad(a_d, (0, pad))


def _unpack_den(den):
    return (den[0] + den[1]).reshape(NPH * 8, 2)[:, 0:1]


def kernel(X, edge_index, edge_weight, fn_W1, fn_b1, fn_W2, fn_b2,
           gat1_W, gat1_as, gat1_ad, gat1_b,
           gat2_W, gat2_as, gat2_ad, gat2_b,
           gc_W, gc_b, gen_W, gen_b, dec_W, dec_b):
    loop = jnp.arange(N, dtype=jnp.int32)
    padi = jnp.full((EP - EA,), N, jnp.int32)
    src_all = jnp.concatenate([edge_index[0], loop, padi])
    dst_all = jnp.concatenate([edge_index[1], loop, padi])
    pk3d = (src_all | (dst_all << 14)).reshape(NWORK, CPT, CH)
    pad = NP - N

    h1, as1, ad1 = _tc1(X, fn_W1, fn_b1, fn_W2, fn_b2, gat1_W, gat1_as, gat1_ad)
    asrc1, adst1 = _shift_and_pad(as1, ad1)
    out1, den1 = _sc_gat(pk3d, asrc1, adst1, h1)

    h2, as2, ad2 = _tc2(out1, _unpack_den(den1), gat1_b, gat2_W, gat2_as, gat2_ad)
    asrc2, adst2 = _shift_and_pad(as2, ad2)
    out2, den2 = _sc_gat(pk3d, asrc2, adst2, h2)

    acc = _tc3(out2, _unpack_den(den2), gat2_b, gc_W, gc_b, gen_W, gen_b,
               dec_W, dec_b, X)
    return acc[0, 0] / float(N * IN_DIM)
